# baseline (speedup 1.0000x reference)
import jax
import jax.numpy as jnp
from jax.experimental import pallas as pl
from jax.experimental.pallas import tpu as pltpu

S = 8192
LANES = 128
ROWS = S // LANES
GRID = 8
RCHUNKS = (S // GRID) // LANES
FLUSH = 32


def _hinge_body(p2r_ref, t2r_ref, p2_ref, t2_ref, out_ref):
    p8t = jnp.swapaxes(p2r_ref[:, :], 0, 1)
    t8t = jnp.swapaxes(t2r_ref[:, :], 0, 1)
    p2b = p2_ref[:, :].astype(jnp.bfloat16)
    t2b = t2_ref[:, :].astype(jnp.bfloat16)
    facc = jnp.zeros((LANES, LANES), jnp.float32)
    for r in range(RCHUNKS):
        aib = (1.0 - p8t[:, r:r + 1]).astype(jnp.bfloat16)
        tib = t8t[:, r:r + 1].astype(jnp.bfloat16)
        for half in range(ROWS // FLUSH):
            sub = jnp.zeros((LANES, LANES), jnp.bfloat16)
            for cc in range(FLUSH):
                c = half * FLUSH + cc
                pj = p2b[c:c + 1, :]
                tj = t2b[c:c + 1, :]
                h = jnp.maximum(aib + pj, jnp.bfloat16(0.0))
                sub = sub + jnp.where(tib > tj, h, jnp.bfloat16(0.0))
            facc = facc + sub.astype(jnp.float32)
    total = jnp.sum(facc, keepdims=True).reshape(1, 1, 1)

    @pl.when(pl.program_id(0) == 0)
    def _():
        out_ref[:, :, :] = jnp.zeros((1, 1, 1), jnp.float32)

    out_ref[:, :, :] = out_ref[:, :, :] + total


def kernel(probs, targets, idx):
    idx = idx.astype(jnp.int32)
    n = probs.shape[0]
    big = jnp.concatenate([probs, targets])
    gidx = jnp.concatenate([idx, idx + n])
    g = big[gidx]
    p2 = g[:S].reshape(ROWS, LANES)
    t2 = g[S:].reshape(ROWS, LANES)
    out = pl.pallas_call(
        _hinge_body,
        grid=(GRID,),
        in_specs=[
            pl.BlockSpec((RCHUNKS, LANES), lambda g: (g, 0)),
            pl.BlockSpec((RCHUNKS, LANES), lambda g: (g, 0)),
            pl.BlockSpec((ROWS, LANES), lambda g: (0, 0)),
            pl.BlockSpec((ROWS, LANES), lambda g: (0, 0)),
        ],
        out_specs=pl.BlockSpec((1, 1, 1), lambda g: (0, 0, 0)),
        out_shape=jax.ShapeDtypeStruct((1, 1, 1), jnp.float32),
        compiler_params=pltpu.CompilerParams(
            dimension_semantics=("arbitrary",)),
    )(p2, t2, p2, t2)
    return out.reshape(())


# speedup vs baseline: 1.1160x; 1.1160x over previous
import jax
import jax.numpy as jnp
from jax.experimental import pallas as pl
from jax.experimental.pallas import tpu as pltpu

S = 8192
LANES = 128
ROWS = S // LANES
GRID = 8
RCHUNKS = (S // GRID) // LANES
FLUSH = 32


def _hinge_body(p2r_ref, t2r_ref, p2_ref, t2_ref, out_ref):
    p8t = jnp.swapaxes(p2r_ref[:, :], 0, 1)
    t8t = jnp.swapaxes(t2r_ref[:, :], 0, 1)
    p2b = p2_ref[:, :].astype(jnp.bfloat16)
    t2b = t2_ref[:, :].astype(jnp.bfloat16)
    facc = jnp.zeros((LANES, LANES), jnp.float32)
    for r in range(RCHUNKS):
        aib = (1.0 - p8t[:, r:r + 1]).astype(jnp.bfloat16)
        tib = t8t[:, r:r + 1].astype(jnp.bfloat16)
        for half in range(ROWS // FLUSH):
            sub = jnp.zeros((LANES, LANES), jnp.bfloat16)
            for cc in range(FLUSH):
                c = half * FLUSH + cc
                pj = p2b[c:c + 1, :]
                tj = t2b[c:c + 1, :]
                h = jnp.maximum(aib + pj, jnp.bfloat16(0.0))
                sub = sub + jnp.where(tib > tj, h, jnp.bfloat16(0.0))
            facc = facc + sub.astype(jnp.float32)
    total = jnp.sum(facc, keepdims=True).reshape(1, 1, 1)

    @pl.when(pl.program_id(0) == 0)
    def _():
        out_ref[:, :, :] = jnp.zeros((1, 1, 1), jnp.float32)

    out_ref[:, :, :] = out_ref[:, :, :] + total


def kernel(probs, targets, idx):
    idx = idx.astype(jnp.int32)
    p = probs[idx]
    t = targets[idx]
    p2 = p.reshape(ROWS, LANES)
    t2 = t.reshape(ROWS, LANES)
    out = pl.pallas_call(
        _hinge_body,
        grid=(GRID,),
        in_specs=[
            pl.BlockSpec((RCHUNKS, LANES), lambda g: (g, 0)),
            pl.BlockSpec((RCHUNKS, LANES), lambda g: (g, 0)),
            pl.BlockSpec((ROWS, LANES), lambda g: (0, 0)),
            pl.BlockSpec((ROWS, LANES), lambda g: (0, 0)),
        ],
        out_specs=pl.BlockSpec((1, 1, 1), lambda g: (0, 0, 0)),
        out_shape=jax.ShapeDtypeStruct((1, 1, 1), jnp.float32),
        compiler_params=pltpu.CompilerParams(
            dimension_semantics=("arbitrary",)),
    )(p2, t2, p2, t2)
    return out.reshape(())
